# Initial kernel scaffold; baseline (speedup 1.0000x reference)
import jax
import jax.numpy as jnp
from jax.experimental import pallas as pl


def kernel(ids, word2vec, W1, b1, W2, b2):
    raise NotImplementedError("write your pallas kernel here")


# speedup vs baseline: 37.7807x; 37.7807x over previous
import functools

import jax
import jax.numpy as jnp
from jax import lax
from jax.experimental import pallas as pl
from jax.experimental.pallas import tpu as pltpu
from jax.experimental.pallas import tpu_sc as plsc

_NC = 2
_NS = 16
_L = 16
_NW = _NC * _NS


def _table_body(wv_ref, w1_ref, b1_ref, w2_ref, b2_ref, out_ref):
    w = jnp.dot(w1_ref[...], w2_ref[...], preferred_element_type=jnp.float32)
    c = jnp.dot(b1_ref[...], w2_ref[...], preferred_element_type=jnp.float32)
    c = c + b2_ref[...]
    out_ref[...] = (
        jnp.dot(wv_ref[...], w, preferred_element_type=jnp.float32) + c
    )


def _build_table(word2vec, W1, b1, W2, b2):
    V, D = word2vec.shape
    H = W1.shape[1]
    BLK = 4000
    assert V % BLK == 0
    grid = V // BLK
    table = pl.pallas_call(
        _table_body,
        grid=(grid,),
        in_specs=[
            pl.BlockSpec((BLK, D), lambda i: (i, 0)),
            pl.BlockSpec((D, H), lambda i: (0, 0)),
            pl.BlockSpec((1, H), lambda i: (0, 0)),
            pl.BlockSpec((H, 1), lambda i: (0, 0)),
            pl.BlockSpec((1, 1), lambda i: (0, 0)),
        ],
        out_specs=pl.BlockSpec((BLK, 1), lambda i: (i, 0)),
        out_shape=jax.ShapeDtypeStruct((V, 1), jnp.float32),
    )(word2vec, W1, b1.reshape(1, H), W2, b2.reshape(1, 1))
    return table.reshape(V)


def _make_pool_kernel(B, S, V):
    BPW = B // _NW
    NCHUNK = 2
    CROWS = BPW // NCHUNK
    assert BPW % (NCHUNK * _L) == 0
    GROUPS = CROWS // _L
    inv_s = 1.0 / S

    mesh = plsc.VectorSubcoreMesh(
        core_axis_name="c", subcore_axis_name="s",
        num_cores=_NC, num_subcores=_NS,
    )

    @functools.partial(
        pl.kernel,
        out_type=jax.ShapeDtypeStruct((B,), jnp.float32),
        mesh=mesh,
        compiler_params=pltpu.CompilerParams(needs_layout_passes=False),
        scratch_types=[
            pltpu.VMEM((V,), jnp.float32),
            pltpu.VMEM((CROWS * S,), jnp.int32),
            pltpu.VMEM((BPW,), jnp.float32),
        ],
    )
    def pool(table_hbm, ids_hbm, out_hbm, table_v, ids_v, out_v):
        wid = lax.axis_index("s") * _NC + lax.axis_index("c")
        base = wid * BPW
        pltpu.sync_copy(table_hbm, table_v)
        iota = lax.iota(jnp.int32, _L)
        lane_off = iota * S
        for chunk in range(NCHUNK):
            pltpu.sync_copy(
                ids_hbm.at[pl.ds((base + chunk * CROWS) * S, CROWS * S)],
                ids_v,
            )

            def group_body(g, carry, _chunk=chunk):
                rowbase = lane_off + g * (_L * S)
                acc = jnp.zeros((_L,), jnp.float32)
                for s in range(S):
                    idv = plsc.load_gather(ids_v, [rowbase + s])
                    acc = acc + plsc.load_gather(table_v, [idv])
                out_v[pl.ds(_chunk * CROWS + g * _L, _L)] = acc * inv_s
                return carry

            lax.fori_loop(0, GROUPS, group_body, 0)
        pltpu.sync_copy(out_v, out_hbm.at[pl.ds(base, BPW)])

    return pool


def kernel(ids, word2vec, W1, b1, W2, b2):
    B, S = ids.shape
    V, _D = word2vec.shape
    table = _build_table(word2vec, W1, b1, W2, b2)
    pool = _make_pool_kernel(B, S, V)
    out = pool(table, ids.reshape(B * S))
    return out.reshape(B, 1)


# speedup vs baseline: 51.4772x; 1.3625x over previous
import functools

import jax
import jax.numpy as jnp
from jax import lax
from jax.experimental import pallas as pl
from jax.experimental.pallas import tpu as pltpu
from jax.experimental.pallas import tpu_sc as plsc

_NC = 2
_NS = 16
_L = 16
_NW = _NC * _NS


def _table_body(wv_ref, w1_ref, b1_ref, w2t_ref, b2_ref, out_ref):
    w_row = jax.lax.dot_general(
        w2t_ref[...], w1_ref[...], (((1,), (1,)), ((), ())),
        preferred_element_type=jnp.float32,
    )
    c = jnp.sum(b1_ref[...] * w2t_ref[...]) + b2_ref[0, 0]
    t_row = jax.lax.dot_general(
        w_row, wv_ref[...], (((1,), (1,)), ((), ())),
        preferred_element_type=jnp.float32,
    )
    out_ref[...] = t_row[0] + c


def _build_table(word2vec, W1, b1, W2, b2):
    V, D = word2vec.shape
    H = W1.shape[1]
    BLK = 4096
    grid = (V + BLK - 1) // BLK
    VP = grid * BLK
    return pl.pallas_call(
        _table_body,
        grid=(grid,),
        in_specs=[
            pl.BlockSpec((BLK, D), lambda i: (i, 0)),
            pl.BlockSpec((D, H), lambda i: (0, 0)),
            pl.BlockSpec((1, H), lambda i: (0, 0)),
            pl.BlockSpec((1, H), lambda i: (0, 0)),
            pl.BlockSpec((1, 1), lambda i: (0, 0)),
        ],
        out_specs=pl.BlockSpec((BLK,), lambda i: (i,)),
        out_shape=jax.ShapeDtypeStruct((VP,), jnp.float32),
    )(word2vec, W1, b1.reshape(1, H), W2.reshape(1, H), b2.reshape(1, 1))


def _make_pool_kernel(B, S, V):
    BPW = B // _NW
    NCHUNK = 2
    CROWS = BPW // NCHUNK
    assert BPW % (NCHUNK * _L) == 0
    GROUPS = CROWS // _L
    inv_s = 1.0 / S

    mesh = plsc.VectorSubcoreMesh(
        core_axis_name="c", subcore_axis_name="s",
        num_cores=_NC, num_subcores=_NS,
    )

    @functools.partial(
        pl.kernel,
        out_type=jax.ShapeDtypeStruct((B,), jnp.float32),
        mesh=mesh,
        compiler_params=pltpu.CompilerParams(needs_layout_passes=False),
        scratch_types=[
            pltpu.VMEM((V,), jnp.float32),
            pltpu.VMEM((CROWS * S,), jnp.int32),
            pltpu.VMEM((BPW,), jnp.float32),
        ],
    )
    def pool(table_hbm, ids_hbm, out_hbm, table_v, ids_v, out_v):
        wid = lax.axis_index("s") * _NC + lax.axis_index("c")
        base = wid * BPW
        pltpu.sync_copy(table_hbm, table_v)
        iota = lax.iota(jnp.int32, _L)
        lane_off = iota * S
        for chunk in range(NCHUNK):
            pltpu.sync_copy(
                ids_hbm.at[pl.ds((base + chunk * CROWS) * S, CROWS * S)],
                ids_v,
            )

            def group_body(g, carry, _chunk=chunk):
                rowbase = lane_off + g * (_L * S)
                acc = jnp.zeros((_L,), jnp.float32)
                for s in range(S):
                    idv = plsc.load_gather(ids_v, [rowbase + s])
                    acc = acc + plsc.load_gather(table_v, [idv])
                out_v[pl.ds(_chunk * CROWS + g * _L, _L)] = acc * inv_s
                return carry

            lax.fori_loop(0, GROUPS, group_body, 0)
        pltpu.sync_copy(out_v, out_hbm.at[pl.ds(base, BPW)])

    return pool


def kernel(ids, word2vec, W1, b1, W2, b2):
    B, S = ids.shape
    V, _D = word2vec.shape
    table = _build_table(word2vec, W1, b1, W2, b2)
    pool = _make_pool_kernel(B, S, table.shape[0])
    out = pool(table, ids.reshape(B * S))
    return out.reshape(B, 1)
